# baseline (speedup 1.0000x reference)
import functools

import numpy as np
import jax
import jax.numpy as jnp
from jax import lax
from jax.experimental import pallas as pl
from jax.experimental.pallas import tpu as pltpu
from jax.experimental.pallas import tpu_sc as plsc

_V = 100000
_E = 32
_S = 20
_B = 1024
_M = 50

_NG = _B * _M
_NW = 32
_GPC = 32
_IDXW = 80
_IR = _GPC * _S // _IDXW
_SROWS = _NG * _S // _IDXW
_QROWS = _B * _S // _IDXW

_HALVES = 2
_NGH = _NG // _HALVES
_BH = _B // _HALVES
_CHUNKS_H = _NGH // (_NW * _GPC)
_HROWS = _SROWS // _HALVES
_QIRH = _QROWS // _HALVES // _NW
_QGH = _QIRH * _IDXW // _S


def _weighted_reduce(rows_v, acc_v, n_groups):
    scale = 4.0 / (_E * _S)
    d0 = lax.iota(jnp.int32, 16).astype(jnp.float32)
    a0 = (d0 - 15.5) * scale
    a1 = (d0 + 0.5) * scale

    def grp_body(g, carry):
        base = g * _S
        t0a = rows_v[base + _S - 1, pl.ds(0, 16)]
        t0b = rows_v[base + _S - 1, pl.ds(16, 16)]
        s1a = jnp.zeros((16,), jnp.float32)
        s1b = jnp.zeros((16,), jnp.float32)
        for s in range(_S - 1):
            r0 = rows_v[base + s, pl.ds(0, 16)]
            r1 = rows_v[base + s, pl.ds(16, 16)]
            t0a = t0a + r0
            t0b = t0b + r1
            w = float(s) - 9.5
            s1a = s1a + w * r0
            s1b = s1b + w * r1
        acc_v[g, pl.ds(0, 16)] = t0a + a0 * s1a
        acc_v[g, pl.ds(16, 16)] = t0b + a1 * s1b
        return carry

    lax.fori_loop(0, n_groups, grp_body, 0)


def _make_sc_table_body(half):
    def body(stories_h, tab_h, out_h,
             idx_v0, idx_v1, rows_v0, rows_v1, acc_v0, acc_v1,
             gsem0, gsem1, osem0, osem1):
        nc = plsc.get_sparse_core_info().num_cores
        wid = lax.axis_index("s") * nc + lax.axis_index("c")
        idx_vs = (idx_v0, idx_v1)
        rows_vs = (rows_v0, rows_v1)
        acc_vs = (acc_v0, acc_v1)
        gsems = (gsem0, gsem1)
        osems = (osem0, osem1)

        grp_base = wid * (_CHUNKS_H * _GPC)
        row_base = half * _HROWS + wid * (_CHUNKS_H * _IR)

        def fire(buf, c):
            rb = row_base + c * _IR
            pltpu.sync_copy(stories_h.at[pl.ds(rb, _IR), :], idx_vs[buf])
            for i in range(_IR):
                pltpu.async_copy(tab_h.at[idx_vs[buf].at[i]],
                                 rows_vs[buf].at[pl.ds(i * _IDXW, _IDXW)],
                                 gsems[buf])

        def wait_gather(buf):
            for i in range(_IR):
                pltpu.make_async_copy(tab_h.at[idx_vs[buf].at[i]],
                                      rows_vs[buf].at[pl.ds(i * _IDXW, _IDXW)],
                                      gsems[buf]).wait()

        def out_copy(buf, c):
            gb = grp_base + c * _GPC
            return pltpu.make_async_copy(
                acc_vs[buf], out_h.at[pl.ds(gb, _GPC), :], osems[buf])

        def step(buf, cc, c):
            @pl.when(cc > 0)
            def _():
                out_copy(buf, c).wait()
            wait_gather(buf)
            nxt = c + 2
            @pl.when(nxt < _CHUNKS_H)
            def _():
                fire(buf, nxt)
            _weighted_reduce(rows_vs[buf], acc_vs[buf], _GPC)
            pltpu.async_copy(acc_vs[buf],
                             out_h.at[pl.ds(grp_base + c * _GPC, _GPC), :],
                             osems[buf])

        fire(0, 0)
        fire(1, 1)

        def loop_body(cc, carry):
            step(0, cc, 2 * cc)
            step(1, cc, 2 * cc + 1)
            return carry

        lax.fori_loop(0, _CHUNKS_H // 2, loop_body, 0)
        last = _CHUNKS_H - 1
        out_copy(0, last - 2).wait()
        wait_gather(0)
        _weighted_reduce(rows_v0, acc_v0, _GPC)
        pltpu.async_copy(acc_v0,
                         out_h.at[pl.ds(grp_base + last * _GPC, _GPC), :],
                         osem0)
        out_copy(0, last).wait()
        out_copy(1, last - 1).wait()

    return body


def _make_sc_query_body(half):
    def body(queries_h, aw_h, outq_h, idx_v0, rows_v0, acc_v0, gsem0):
        nc = plsc.get_sparse_core_info().num_cores
        wid = lax.axis_index("s") * nc + lax.axis_index("c")
        qrb = half * (_QROWS // _HALVES) + wid * _QIRH
        pltpu.sync_copy(queries_h.at[pl.ds(qrb, _QIRH), :], idx_v0)
        for i in range(_QIRH):
            pltpu.async_copy(aw_h.at[idx_v0.at[i]],
                             rows_v0.at[pl.ds(i * _IDXW, _IDXW)], gsem0)
        for i in range(_QIRH):
            pltpu.make_async_copy(aw_h.at[idx_v0.at[i]],
                                  rows_v0.at[pl.ds(i * _IDXW, _IDXW)],
                                  gsem0).wait()
        _weighted_reduce(rows_v0, acc_v0, _QGH)
        pltpu.sync_copy(acc_v0, outq_h.at[pl.ds(wid * _QGH, _QGH), :])

    return body


def _sc_mesh():
    return plsc.VectorSubcoreMesh(core_axis_name="c", subcore_axis_name="s")


def _sc_table(stories, table, half):
    f = functools.partial(
        pl.kernel,
        mesh=_sc_mesh(),
        compiler_params=pltpu.CompilerParams(use_tc_tiling_on_sc=False),
        out_type=jax.ShapeDtypeStruct((_NGH, _E), jnp.float32),
        scratch_types=[
            pltpu.VMEM((_IR, _IDXW), jnp.int32),
            pltpu.VMEM((_IR, _IDXW), jnp.int32),
            pltpu.VMEM((_GPC * _S, _E), jnp.float32),
            pltpu.VMEM((_GPC * _S, _E), jnp.float32),
            pltpu.VMEM((_GPC, _E), jnp.float32),
            pltpu.VMEM((_GPC, _E), jnp.float32),
            pltpu.SemaphoreType.DMA,
            pltpu.SemaphoreType.DMA,
            pltpu.SemaphoreType.DMA,
            pltpu.SemaphoreType.DMA,
        ],
    )(_make_sc_table_body(half))
    return f(stories, table)


def _sc_query(queries, A_w, half):
    f = functools.partial(
        pl.kernel,
        mesh=_sc_mesh(),
        compiler_params=pltpu.CompilerParams(use_tc_tiling_on_sc=False),
        out_type=jax.ShapeDtypeStruct((_BH, _E), jnp.float32),
        scratch_types=[
            pltpu.VMEM((_QIRH, _IDXW), jnp.int32),
            pltpu.VMEM((_QIRH * _IDXW, _E), jnp.float32),
            pltpu.VMEM((_QGH, _E), jnp.float32),
            pltpu.SemaphoreType.DMA,
        ],
    )(_make_sc_query_body(half))
    return f(queries, A_w)


_ME = _M * _E
_MP = 56


def _hops_body(u0_ref, ga_ref, g0_ref, g1_ref, g2_ref, out_ref):
    jmod = jax.lax.broadcasted_iota(jnp.int32, (_E, _ME), 1) % _E
    drow = jax.lax.broadcasted_iota(jnp.int32, (_E, _ME), 0)
    t_mat = jnp.where(jmod == drow, 1.0, 0.0)
    jdiv = jax.lax.broadcasted_iota(jnp.int32, (_ME, _MP), 0) // _E
    mcol = jax.lax.broadcasted_iota(jnp.int32, (_ME, _MP), 1)
    s_mat = jnp.where(jdiv == mcol, 1.0, 0.0)
    mvalid = jax.lax.broadcasted_iota(jnp.int32, (1, _MP), 1) < _M

    u = u0_ref[...]

    hi = jax.lax.Precision.HIGHEST

    def hop(u, x_att, x_out):
        ut = jnp.dot(u, t_mat, precision=hi)
        d = jnp.dot(x_att * ut, s_mat, precision=hi)
        d = jnp.where(mvalid, d, -1e30)
        d = d - jnp.max(d, axis=-1, keepdims=True)
        p = jnp.exp(d)
        p = p / jnp.sum(p, axis=-1, keepdims=True)
        prep = jnp.dot(p, s_mat.T, precision=hi)
        o = jnp.dot(x_out * prep, t_mat.T, precision=hi)
        return u + o

    u = hop(u, ga_ref[...], g0_ref[...])
    u = hop(u, g0_ref[...], g1_ref[...])
    u = hop(u, g1_ref[...], g2_ref[...])
    out_ref[...] = u


def _hops(u0, ga, g0, g1, g2):
    bb = 128
    grid = _BH // bb
    spec2 = pl.BlockSpec((bb, _E), lambda i: (i, 0))
    specp = pl.BlockSpec((bb, _ME), lambda i: (i, 0))
    return pl.pallas_call(
        _hops_body,
        grid=(grid,),
        in_specs=[spec2, specp, specp, specp, specp],
        out_specs=spec2,
        out_shape=jax.ShapeDtypeStruct((_BH, _E), jnp.float32),
    )(u0, ga, g0, g1, g2)


def _logits_body(c2t_ref, u_ref, out_ref):
    out_ref[...] = lax.dot_general(
        c2t_ref[...], u_ref[...],
        (((0,), (1,)), ((), ())),
        preferred_element_type=jnp.float32,
    )


def _logits_t(u, c2t):
    vb = 2048
    grid = pl.cdiv(_V, vb)
    return pl.pallas_call(
        _logits_body,
        grid=(grid,),
        in_specs=[
            pl.BlockSpec((_E, vb), lambda i: (0, i)),
            pl.BlockSpec((_B, _E), lambda i: (0, 0)),
        ],
        out_specs=pl.BlockSpec((vb, _B), lambda i: (i, 0)),
        out_shape=jax.ShapeDtypeStruct((_V, _B), jnp.float32),
    )(c2t, u)


def kernel(stories, queries, A_w, C0_w, C1_w, C2_w):
    stories = stories.astype(jnp.int32).reshape(_SROWS, _IDXW)
    queries = queries.astype(jnp.int32).reshape(_QROWS, _IDXW)

    u3s = []
    for half in range(_HALVES):
        u0 = _sc_query(queries, A_w, half)
        ga = _sc_table(stories, A_w, half)
        g0 = _sc_table(stories, C0_w, half)
        g1 = _sc_table(stories, C1_w, half)
        g2 = _sc_table(stories, C2_w, half)
        u3s.append(_hops(
            u0,
            ga.reshape(_BH, _ME),
            g0.reshape(_BH, _ME),
            g1.reshape(_BH, _ME),
            g2.reshape(_BH, _ME),
        ))

    u3 = jnp.concatenate(u3s, axis=0)
    return _logits_t(u3, C2_w.T).T


# speedup vs baseline: 1.1106x; 1.1106x over previous
import functools

import numpy as np
import jax
import jax.numpy as jnp
from jax import lax
from jax.experimental import pallas as pl
from jax.experimental.pallas import tpu as pltpu
from jax.experimental.pallas import tpu_sc as plsc

_V = 100000
_E = 32
_S = 20
_B = 1024
_M = 50

_NG = _B * _M
_NW = 32
_GPC = 32
_IDXW = 80
_IR = _GPC * _S // _IDXW
_SROWS = _NG * _S // _IDXW
_QROWS = _B * _S // _IDXW

_HALVES = 2
_NGH = _NG // _HALVES
_BH = _B // _HALVES
_CHUNKS_H = _NGH // (_NW * _GPC)
_HROWS = _SROWS // _HALVES
_QIRH = _QROWS // _HALVES // _NW
_QGH = _QIRH * _IDXW // _S


def _weighted_reduce(rows_v, acc_v, n_groups):
    scale = 4.0 / (_E * _S)
    d0 = lax.iota(jnp.int32, 16).astype(jnp.float32)
    a0 = (d0 - 15.5) * scale
    a1 = (d0 + 0.5) * scale

    def grp_body(g, carry):
        base = g * _S
        t0a = rows_v[base + _S - 1, pl.ds(0, 16)]
        t0b = rows_v[base + _S - 1, pl.ds(16, 16)]
        s1a = jnp.zeros((16,), jnp.float32)
        s1b = jnp.zeros((16,), jnp.float32)
        for s in range(_S - 1):
            r0 = rows_v[base + s, pl.ds(0, 16)]
            r1 = rows_v[base + s, pl.ds(16, 16)]
            t0a = t0a + r0
            t0b = t0b + r1
            w = float(s) - 9.5
            s1a = s1a + w * r0
            s1b = s1b + w * r1
        acc_v[g, pl.ds(0, 16)] = t0a + a0 * s1a
        acc_v[g, pl.ds(16, 16)] = t0b + a1 * s1b
        return carry

    lax.fori_loop(0, n_groups, grp_body, 0)


def _make_sc_table_body(half):
    def body(stories_h, tab_h, out_h,
             idx_v0, idx_v1, rows_v0, rows_v1, acc_v0, acc_v1,
             gsem0, gsem1, osem0, osem1):
        nc = plsc.get_sparse_core_info().num_cores
        wid = lax.axis_index("s") * nc + lax.axis_index("c")
        idx_vs = (idx_v0, idx_v1)
        rows_vs = (rows_v0, rows_v1)
        acc_vs = (acc_v0, acc_v1)
        gsems = (gsem0, gsem1)
        osems = (osem0, osem1)

        grp_base = wid * (_CHUNKS_H * _GPC)
        row_base = half * _HROWS + wid * (_CHUNKS_H * _IR)

        def fire(buf, c):
            rb = row_base + c * _IR
            pltpu.sync_copy(stories_h.at[pl.ds(rb, _IR), :], idx_vs[buf])
            for i in range(_IR):
                pltpu.async_copy(tab_h.at[idx_vs[buf].at[i]],
                                 rows_vs[buf].at[pl.ds(i * _IDXW, _IDXW)],
                                 gsems[buf])

        def wait_gather(buf):
            for i in range(_IR):
                pltpu.make_async_copy(tab_h.at[idx_vs[buf].at[i]],
                                      rows_vs[buf].at[pl.ds(i * _IDXW, _IDXW)],
                                      gsems[buf]).wait()

        def out_copy(buf, c):
            gb = grp_base + c * _GPC
            return pltpu.make_async_copy(
                acc_vs[buf], out_h.at[pl.ds(gb, _GPC), :], osems[buf])

        def step(buf, cc, c):
            @pl.when(cc > 0)
            def _():
                out_copy(buf, c).wait()
            wait_gather(buf)
            nxt = c + 2
            @pl.when(nxt < _CHUNKS_H)
            def _():
                fire(buf, nxt)
            _weighted_reduce(rows_vs[buf], acc_vs[buf], _GPC)
            pltpu.async_copy(acc_vs[buf],
                             out_h.at[pl.ds(grp_base + c * _GPC, _GPC), :],
                             osems[buf])

        fire(0, 0)
        fire(1, 1)

        def loop_body(cc, carry):
            step(0, cc, 2 * cc)
            step(1, cc, 2 * cc + 1)
            return carry

        lax.fori_loop(0, _CHUNKS_H // 2, loop_body, 0)
        last = _CHUNKS_H - 1
        out_copy(0, last - 2).wait()
        wait_gather(0)
        _weighted_reduce(rows_v0, acc_v0, _GPC)
        pltpu.async_copy(acc_v0,
                         out_h.at[pl.ds(grp_base + last * _GPC, _GPC), :],
                         osem0)
        out_copy(0, last).wait()
        out_copy(1, last - 1).wait()

    return body


def _make_sc_query_body(half):
    def body(queries_h, aw_h, outq_h, idx_v0, rows_v0, acc_v0, gsem0):
        nc = plsc.get_sparse_core_info().num_cores
        wid = lax.axis_index("s") * nc + lax.axis_index("c")
        qrb = half * (_QROWS // _HALVES) + wid * _QIRH
        pltpu.sync_copy(queries_h.at[pl.ds(qrb, _QIRH), :], idx_v0)
        for i in range(_QIRH):
            pltpu.async_copy(aw_h.at[idx_v0.at[i]],
                             rows_v0.at[pl.ds(i * _IDXW, _IDXW)], gsem0)
        for i in range(_QIRH):
            pltpu.make_async_copy(aw_h.at[idx_v0.at[i]],
                                  rows_v0.at[pl.ds(i * _IDXW, _IDXW)],
                                  gsem0).wait()
        _weighted_reduce(rows_v0, acc_v0, _QGH)
        pltpu.sync_copy(acc_v0, outq_h.at[pl.ds(wid * _QGH, _QGH), :])

    return body


def _sc_mesh():
    return plsc.VectorSubcoreMesh(core_axis_name="c", subcore_axis_name="s")


def _sc_table(stories, table, half):
    f = functools.partial(
        pl.kernel,
        mesh=_sc_mesh(),
        compiler_params=pltpu.CompilerParams(use_tc_tiling_on_sc=False),
        out_type=jax.ShapeDtypeStruct((_NGH, _E), jnp.float32),
        scratch_types=[
            pltpu.VMEM((_IR, _IDXW), jnp.int32),
            pltpu.VMEM((_IR, _IDXW), jnp.int32),
            pltpu.VMEM((_GPC * _S, _E), jnp.float32),
            pltpu.VMEM((_GPC * _S, _E), jnp.float32),
            pltpu.VMEM((_GPC, _E), jnp.float32),
            pltpu.VMEM((_GPC, _E), jnp.float32),
            pltpu.SemaphoreType.DMA,
            pltpu.SemaphoreType.DMA,
            pltpu.SemaphoreType.DMA,
            pltpu.SemaphoreType.DMA,
        ],
    )(_make_sc_table_body(half))
    return f(stories, table)


def _sc_query(queries, A_w, half):
    f = functools.partial(
        pl.kernel,
        mesh=_sc_mesh(),
        compiler_params=pltpu.CompilerParams(use_tc_tiling_on_sc=False),
        out_type=jax.ShapeDtypeStruct((_BH, _E), jnp.float32),
        scratch_types=[
            pltpu.VMEM((_QIRH, _IDXW), jnp.int32),
            pltpu.VMEM((_QIRH * _IDXW, _E), jnp.float32),
            pltpu.VMEM((_QGH, _E), jnp.float32),
            pltpu.SemaphoreType.DMA,
        ],
    )(_make_sc_query_body(half))
    return f(queries, A_w)


_ME = _M * _E
_MP = 56


def _hops_body(u0_ref, ga_ref, g0_ref, g1_ref, g2_ref, out_ref):
    jmod = jax.lax.broadcasted_iota(jnp.int32, (_E, _ME), 1) % _E
    drow = jax.lax.broadcasted_iota(jnp.int32, (_E, _ME), 0)
    t_mat = jnp.where(jmod == drow, 1.0, 0.0)
    jdiv = jax.lax.broadcasted_iota(jnp.int32, (_ME, _MP), 0) // _E
    mcol = jax.lax.broadcasted_iota(jnp.int32, (_ME, _MP), 1)
    s_mat = jnp.where(jdiv == mcol, 1.0, 0.0)
    mvalid = jax.lax.broadcasted_iota(jnp.int32, (1, _MP), 1) < _M

    u = u0_ref[...]

    hi = None

    def hop(u, x_att, x_out):
        ut = jnp.dot(u, t_mat, precision=hi)
        d = jnp.dot(x_att * ut, s_mat, precision=hi)
        d = jnp.where(mvalid, d, -1e30)
        d = d - jnp.max(d, axis=-1, keepdims=True)
        p = jnp.exp(d)
        p = p / jnp.sum(p, axis=-1, keepdims=True)
        prep = jnp.dot(p, s_mat.T, precision=hi)
        o = jnp.dot(x_out * prep, t_mat.T, precision=hi)
        return u + o

    u = hop(u, ga_ref[...], g0_ref[...])
    u = hop(u, g0_ref[...], g1_ref[...])
    u = hop(u, g1_ref[...], g2_ref[...])
    out_ref[...] = u


def _hops(u0, ga, g0, g1, g2):
    bb = 128
    grid = _BH // bb
    spec2 = pl.BlockSpec((bb, _E), lambda i: (i, 0))
    specp = pl.BlockSpec((bb, _ME), lambda i: (i, 0))
    return pl.pallas_call(
        _hops_body,
        grid=(grid,),
        in_specs=[spec2, specp, specp, specp, specp],
        out_specs=spec2,
        out_shape=jax.ShapeDtypeStruct((_BH, _E), jnp.float32),
    )(u0, ga, g0, g1, g2)


def _logits_body(c2t_ref, u_ref, out_ref):
    out_ref[...] = lax.dot_general(
        c2t_ref[...], u_ref[...],
        (((0,), (1,)), ((), ())),
        preferred_element_type=jnp.float32,
    )


def _logits_t(u, c2t):
    vb = 2048
    grid = pl.cdiv(_V, vb)
    return pl.pallas_call(
        _logits_body,
        grid=(grid,),
        in_specs=[
            pl.BlockSpec((_E, vb), lambda i: (0, i)),
            pl.BlockSpec((_B, _E), lambda i: (0, 0)),
        ],
        out_specs=pl.BlockSpec((vb, _B), lambda i: (i, 0)),
        out_shape=jax.ShapeDtypeStruct((_V, _B), jnp.float32),
    )(c2t, u)


def kernel(stories, queries, A_w, C0_w, C1_w, C2_w):
    stories = stories.astype(jnp.int32).reshape(_SROWS, _IDXW)
    queries = queries.astype(jnp.int32).reshape(_QROWS, _IDXW)

    u3s = []
    for half in range(_HALVES):
        u0 = _sc_query(queries, A_w, half)
        ga = _sc_table(stories, A_w, half)
        g0 = _sc_table(stories, C0_w, half)
        g1 = _sc_table(stories, C1_w, half)
        g2 = _sc_table(stories, C2_w, half)
        u3s.append(_hops(
            u0,
            ga.reshape(_BH, _ME),
            g0.reshape(_BH, _ME),
            g1.reshape(_BH, _ME),
            g2.reshape(_BH, _ME),
        ))

    u3 = jnp.concatenate(u3s, axis=0)
    return _logits_t(u3, C2_w.T).T
